# baseline (speedup 1.0000x reference)
import functools

import jax
import jax.numpy as jnp
from jax import lax
from jax.experimental import pallas as pl
from jax.experimental.pallas import tpu as pltpu
from jax.experimental.pallas import tpu_sc as plsc

_N = 10000
_E = 320000
_H = 128
_L = 4
_G = 400

_AC = [119, 4, 12, 12, 10, 6, 6, 2, 2]
_ACP = [120, 8, 16, 16, 16, 8, 8, 8, 8]
_AOFF = [0, 120, 128, 144, 160, 176, 184, 192, 200]
_ATOT = 208
_ECP = [8, 8, 8]
_EOFF = [0, 8, 16]
_ETOT = 24

_NC = 2
_NS = 16
_NW = _NC * _NS
_EPW = _E // _NW
_CH = 80
_NCHUNK = _EPW // _CH
_ZR = 40
_NZCH = _N // _ZR
_ZPT = -(-_NZCH // _NS)


def _gelu(v):
    return 0.5 * v * (1.0 + lax.erf(v * 0.7071067811865476))


def _atom_body(x_ref, emb_ref, w_ref, b_ref, o_ref):
    xb = x_ref[...]
    acc = jnp.zeros((xb.shape[0], _H), jnp.float32) + b_ref[...]
    for i in range(9):
        cp, off = _ACP[i], _AOFF[i]
        ti = jnp.dot(emb_ref[off:off + cp, :], w_ref[48 * i:48 * (i + 1), :],
                     preferred_element_type=jnp.float32)
        oh = (xb[:, i:i + 1] == lax.broadcasted_iota(
            jnp.int32, (xb.shape[0], cp), 1)).astype(jnp.float32)
        acc = acc + jnp.dot(oh, ti, preferred_element_type=jnp.float32)
    o_ref[...] = _gelu(acc)


def _edge_body(a_ref, emb_ref, w_ref, b_ref, o_ref):
    ab = a_ref[...]
    acc = jnp.zeros((ab.shape[0], _H), jnp.float32) + b_ref[...]
    for j in range(3):
        cp, off = _ECP[j], _EOFF[j]
        tj = jnp.dot(emb_ref[off:off + cp, :], w_ref[...],
                     preferred_element_type=jnp.float32)
        oh = (ab[:, j:j + 1] == lax.broadcasted_iota(
            jnp.int32, (ab.shape[0], cp), 1)).astype(jnp.float32)
        acc = acc + jnp.dot(oh, tj, preferred_element_type=jnp.float32)
    o_ref[...] = _gelu(acc)


def _mlp_body(h_ref, agg_ref, w1_ref, b1_ref, w2_ref, b2_ref, g_ref, be_ref,
              sc_ref, o_ref):
    hb = h_ref[...]
    outv = hb * sc_ref[...] + agg_ref[0] + agg_ref[1]
    t = jnp.dot(_gelu(jnp.dot(outv, w1_ref[...],
                              preferred_element_type=jnp.float32) + b1_ref[...]),
                w2_ref[...], preferred_element_type=jnp.float32) + b2_ref[...]
    m = jnp.mean(t, axis=-1, keepdims=True)
    var = jnp.mean((t - m) ** 2, axis=-1, keepdims=True)
    tn = (t - m) / jnp.sqrt(var + 1e-5) * g_ref[...] + be_ref[...]
    o_ref[...] = hb + _gelu(tn)


def _pool_body(h_ref, b_ref, wg1_ref, bg1_ref, wg2_ref, bg2_ref, wr_ref,
               br_ref, wp1_ref, bp1_ref, wp2_ref, bp2_ref, o_ref):
    hb = h_ref[...]
    bb = b_ref[...]
    hg = _gelu(jnp.dot(hb, wg1_ref[...],
                       preferred_element_type=jnp.float32) + bg1_ref[...])
    gate = jnp.dot(hg, wg2_ref[...],
                   preferred_element_type=jnp.float32) + bg2_ref[...]
    mb = bb == lax.broadcasted_iota(jnp.int32, (_N, _G), 1)
    mf = mb.astype(jnp.float32)
    gmax = jnp.max(jnp.where(mb, gate, -1e30), axis=0, keepdims=True)
    gmaxb = lax.dot_general(mf, gmax, (((1,), (1,)), ((), ())),
                            preferred_element_type=jnp.float32)
    att = jnp.exp(gate - gmaxb)
    den = lax.dot_general(mf, att, (((0,), (0,)), ((), ())),
                          preferred_element_type=jnp.float32)
    denb = jnp.dot(mf, den, preferred_element_type=jnp.float32)
    w = att / (denb + 1e-16)
    g = lax.dot_general(mf, w * hb, (((0,), (0,)), ((), ())),
                        preferred_element_type=jnp.float32)
    g = _gelu(jnp.dot(g, wr_ref[...],
                      preferred_element_type=jnp.float32) + br_ref[...])
    g = _gelu(jnp.dot(g, wp1_ref[...],
                      preferred_element_type=jnp.float32) + bp1_ref[...])
    g = jnp.dot(g, wp2_ref[...],
                preferred_element_type=jnp.float32) + bp2_ref[...]
    nrm = jnp.sqrt(jnp.sum(g * g, axis=-1, keepdims=True))
    o_ref[...] = g / (nrm + 1e-12)


def _full(shape):
    return pl.BlockSpec(shape, lambda *_: tuple(0 for _ in shape))


def _sc_body(h_hbm, e_hbm, src_hbm, dst_hbm, out_hbm,
             src_v0, src_v1, dst_v0, dst_v1, hsrc_v0, hsrc_v1, ev0, ev1,
             zv, agg_sh,
             semI0, semI1, semE0, semE1, semG0, semG1, semS0, semS1):
    c = lax.axis_index("c")
    s = lax.axis_index("s")
    wid = s * _NC + c
    zero16 = jnp.zeros((16,), jnp.float32)

    src_v = (src_v0, src_v1)
    dst_v = (dst_v0, dst_v1)
    hsrc_v = (hsrc_v0, hsrc_v1)
    ev = (ev0, ev1)
    semI = (semI0, semI1)
    semE = (semE0, semE1)
    semG = (semG0, semG1)
    semS = (semS0, semS1)

    def zrow(r, carry):
        for j in range(8):
            zv[r, pl.ds(j * 16, 16)] = zero16
        return carry
    lax.fori_loop(0, _ZR, zrow, 0)
    for t in range(_ZPT):
        k = s + t * _NS

        @pl.when(k < _NZCH)
        def _zero_chunk(k=k):
            pltpu.sync_copy(zv, agg_sh.at[pl.ds(k * _ZR, _ZR)])
    plsc.subcore_barrier()

    ebase = wid * _EPW

    def _step(k, b, first):
        bn = 1 - b
        if not first:
            pltpu.make_async_copy(ev[bn], agg_sh.at[dst_v[bn]],
                                  semS[bn]).wait()
        kn = k + 1
        basen = ebase + kn * _CH

        @pl.when(kn < _NCHUNK)
        def _prefetch():
            pltpu.async_copy(src_hbm.at[pl.ds(basen, _CH)], src_v[bn],
                             semI[bn])
            pltpu.async_copy(dst_hbm.at[pl.ds(basen, _CH)], dst_v[bn],
                             semI[bn])
            pltpu.async_copy(e_hbm.at[pl.ds(basen, _CH)], ev[bn], semE[bn])

        pltpu.make_async_copy(h_hbm.at[src_v[b]], hsrc_v[b], semG[b]).wait()
        pltpu.make_async_copy(e_hbm.at[pl.ds(ebase + k * _CH, _CH)], ev[b],
                              semE[b]).wait()

        @plsc.parallel_loop(0, _CH, step=1, unroll=2)
        def _row(r):
            for j in range(8):
                sl = pl.ds(j * 16, 16)
                ev[b][r, sl] = jnp.maximum(ev[b][r, sl] + hsrc_v[b][r, sl],
                                           0.0)
        pltpu.async_copy(ev[b], agg_sh.at[dst_v[b]], semS[b], add=True)

        @pl.when(kn < _NCHUNK)
        def _next_gather():
            pltpu.make_async_copy(src_hbm.at[pl.ds(basen, _CH)], src_v[bn],
                                  semI[bn]).wait()
            pltpu.make_async_copy(dst_hbm.at[pl.ds(basen, _CH)], dst_v[bn],
                                  semI[bn]).wait()
            pltpu.async_copy(h_hbm.at[src_v[bn]], hsrc_v[bn], semG[bn])

    pltpu.async_copy(src_hbm.at[pl.ds(ebase, _CH)], src_v[0], semI[0])
    pltpu.async_copy(dst_hbm.at[pl.ds(ebase, _CH)], dst_v[0], semI[0])
    pltpu.async_copy(e_hbm.at[pl.ds(ebase, _CH)], ev[0], semE[0])
    pltpu.make_async_copy(src_hbm.at[pl.ds(ebase, _CH)], src_v[0],
                          semI[0]).wait()
    pltpu.make_async_copy(dst_hbm.at[pl.ds(ebase, _CH)], dst_v[0],
                          semI[0]).wait()
    pltpu.async_copy(h_hbm.at[src_v[0]], hsrc_v[0], semG[0])

    _step(0, 0, True)

    def pair(o, carry):
        _step(1 + 2 * o, 1, False)
        _step(2 + 2 * o, 0, False)
        return carry
    lax.fori_loop(0, (_NCHUNK - 1) // 2, pair, 0)
    pltpu.make_async_copy(ev[0], agg_sh.at[dst_v[0]], semS[0]).wait()
    plsc.subcore_barrier()

    for t in range(_ZPT):
        k = s + t * _NS

        @pl.when(k < _NZCH)
        def _copy_chunk(k=k):
            pltpu.sync_copy(agg_sh.at[pl.ds(k * _ZR, _ZR)],
                            out_hbm.at[c, pl.ds(k * _ZR, _ZR)])


def _sc_agg(h, e, src, dst):
    mesh = plsc.VectorSubcoreMesh(core_axis_name="c", subcore_axis_name="s")
    return pl.kernel(
        _sc_body,
        out_type=jax.ShapeDtypeStruct((_NC, _N, _H), jnp.float32),
        mesh=mesh,
        scratch_types=(
            [pltpu.VMEM((_CH,), jnp.int32)] * 4
            + [pltpu.VMEM((_CH, _H), jnp.float32)] * 4
            + [pltpu.VMEM((_ZR, _H), jnp.float32),
               pltpu.VMEM_SHARED((_N, _H), jnp.float32)]
            + [pltpu.SemaphoreType.DMA] * 8
        ),
    )(h, e, src, dst)


def kernel(x, edge_index, edge_attr, batch, atom_emb0, atom_emb1, atom_emb2,
           atom_emb3, atom_emb4, atom_emb5, atom_emb6, atom_emb7, atom_emb8,
           edge_emb0, edge_emb1, edge_emb2, W_atom, b_atom, W_edge, b_edge,
           eps, W1, b1, W2, b2, ln_g, ln_b, Wg1, bg1, Wg2, bg2, Wr, br,
           Wp1, bp1, Wp2, bp2):
    atom_embs = [atom_emb0, atom_emb1, atom_emb2, atom_emb3, atom_emb4,
                 atom_emb5, atom_emb6, atom_emb7, atom_emb8]
    emb_a = jnp.concatenate(
        [jnp.pad(t, ((0, p - t.shape[0]), (0, 0)))
         for t, p in zip(atom_embs, _ACP)], axis=0)
    emb_e = jnp.concatenate(
        [jnp.pad(t, ((0, p - t.shape[0]), (0, 0)))
         for t, p in zip([edge_emb0, edge_emb1, edge_emb2], _ECP)], axis=0)

    h = pl.pallas_call(
        _atom_body,
        grid=(10,),
        in_specs=[pl.BlockSpec((_N // 10, 9), lambda i: (i, 0)),
                  _full((_ATOT, 48)), _full((9 * 48, _H)), _full((1, _H))],
        out_specs=pl.BlockSpec((_N // 10, _H), lambda i: (i, 0)),
        out_shape=jax.ShapeDtypeStruct((_N, _H), jnp.float32),
    )(x, emb_a, W_atom, b_atom.reshape(1, -1))

    e = pl.pallas_call(
        _edge_body,
        grid=(80,),
        in_specs=[pl.BlockSpec((_E // 80, 3), lambda i: (i, 0)),
                  _full((_ETOT, 48)), _full((48, _H)), _full((1, _H))],
        out_specs=pl.BlockSpec((_E // 80, _H), lambda i: (i, 0)),
        out_shape=jax.ShapeDtypeStruct((_E, _H), jnp.float32),
    )(edge_attr, emb_e, W_edge, b_edge.reshape(1, -1))

    src = edge_index[0]
    dst = edge_index[1]

    mlp = pl.pallas_call(
        _mlp_body,
        grid=(10,),
        in_specs=[pl.BlockSpec((_N // 10, _H), lambda i: (i, 0)),
                  pl.BlockSpec((_NC, _N // 10, _H), lambda i: (0, i, 0)),
                  _full((_H, _H)), _full((1, _H)), _full((_H, _H)),
                  _full((1, _H)), _full((1, _H)), _full((1, _H)),
                  _full((1, 1))],
        out_specs=pl.BlockSpec((_N // 10, _H), lambda i: (i, 0)),
        out_shape=jax.ShapeDtypeStruct((_N, _H), jnp.float32),
    )

    for l in range(_L):
        agg = _sc_agg(h, e, src, dst)
        scale = (1.0 + eps[l]).reshape(1, 1)
        h = mlp(h, agg, W1[l], b1[l].reshape(1, -1), W2[l],
                b2[l].reshape(1, -1), ln_g[l].reshape(1, -1),
                ln_b[l].reshape(1, -1), scale)

    out = pl.pallas_call(
        _pool_body,
        in_specs=[_full((_N, _H)), _full((_N, 1)), _full((_H, _H // 2)),
                  _full((1, _H // 2)), _full((_H // 2, 1)), _full((1, 1)),
                  _full((_H, _H)), _full((1, _H)), _full((_H, _H)),
                  _full((1, _H)), _full((_H, _H)), _full((1, _H))],
        out_specs=_full((_G, _H)),
        out_shape=jax.ShapeDtypeStruct((_G, _H), jnp.float32),
    )(h, batch.reshape(-1, 1), Wg1, bg1.reshape(1, -1), Wg2,
      bg2.reshape(1, -1), Wr, br.reshape(1, -1), Wp1, bp1.reshape(1, -1),
      Wp2, bp2.reshape(1, -1))
    return out


# speedup vs baseline: 1.0509x; 1.0509x over previous
import functools

import jax
import jax.numpy as jnp
from jax import lax
from jax.experimental import pallas as pl
from jax.experimental.pallas import tpu as pltpu
from jax.experimental.pallas import tpu_sc as plsc

_N = 10000
_E = 320000
_H = 128
_L = 4
_G = 400
_HW = _H // 2

_AC = [119, 4, 12, 12, 10, 6, 6, 2, 2]
_ACP = [120, 8, 16, 16, 16, 8, 8, 8, 8]
_AOFF = [0, 120, 128, 144, 160, 176, 184, 192, 200]
_ATOT = 208
_ECP = [8, 8, 8]
_EOFF = [0, 8, 16]
_ETOT = 24

_NC = 2
_NS = 16
_NW = _NC * _NS
_EPW = _E // _NW
_CH = 80
_NCHUNK = _EPW // _CH
_ZR = 40
_NZCH = _N // _ZR
_ZPT = -(-_NZCH // _NS)


def _gelu(v):
    return 0.5 * v * (1.0 + lax.erf(v * 0.7071067811865476))


def _pack_half(v):
    u = lax.bitcast_convert_type(v, jnp.int32)
    rnd = ((u >> 16) & 1) + 32767
    ub = (u + rnd) >> 16
    return (ub[:, :_HW] & 65535) | (ub[:, _HW:] << 16)


def _atom_body(x_ref, emb_ref, w_ref, b_ref, o_ref):
    xb = x_ref[...]
    acc = jnp.zeros((xb.shape[0], _H), jnp.float32) + b_ref[...]
    for i in range(9):
        cp, off = _ACP[i], _AOFF[i]
        ti = jnp.dot(emb_ref[off:off + cp, :], w_ref[48 * i:48 * (i + 1), :],
                     preferred_element_type=jnp.float32)
        oh = (xb[:, i:i + 1] == lax.broadcasted_iota(
            jnp.int32, (xb.shape[0], cp), 1)).astype(jnp.float32)
        acc = acc + jnp.dot(oh, ti, preferred_element_type=jnp.float32)
    o_ref[...] = _gelu(acc)


def _edge_body(a_ref, emb_ref, w_ref, b_ref, o_ref):
    ab = a_ref[...]
    acc = jnp.zeros((ab.shape[0], _H), jnp.float32) + b_ref[...]
    for j in range(3):
        cp, off = _ECP[j], _EOFF[j]
        tj = jnp.dot(emb_ref[off:off + cp, :], w_ref[...],
                     preferred_element_type=jnp.float32)
        oh = (ab[:, j:j + 1] == lax.broadcasted_iota(
            jnp.int32, (ab.shape[0], cp), 1)).astype(jnp.float32)
        acc = acc + jnp.dot(oh, tj, preferred_element_type=jnp.float32)
    p = _pack_half(_gelu(acc))
    p3 = p.reshape(p.shape[0] // _CH, _CH, _HW)
    pp = jnp.concatenate([p3[:, :_CH // 2, :], p3[:, _CH // 2:, :]], axis=2)
    o_ref[...] = pp.reshape(p.shape[0] // 2, _H)


def _mlp_body(h_ref, agg_ref, w1_ref, b1_ref, w2_ref, b2_ref, g_ref, be_ref,
              sc_ref, o_ref):
    hb = h_ref[...]
    outv = hb * sc_ref[...] + agg_ref[0] + agg_ref[1]
    t = jnp.dot(_gelu(jnp.dot(outv, w1_ref[...],
                              preferred_element_type=jnp.float32) + b1_ref[...]),
                w2_ref[...], preferred_element_type=jnp.float32) + b2_ref[...]
    m = jnp.mean(t, axis=-1, keepdims=True)
    var = jnp.mean((t - m) ** 2, axis=-1, keepdims=True)
    tn = (t - m) / jnp.sqrt(var + 1e-5) * g_ref[...] + be_ref[...]
    o_ref[...] = hb + _gelu(tn)


def _pool_body(h_ref, b_ref, wg1_ref, bg1_ref, wg2_ref, bg2_ref, wr_ref,
               br_ref, wp1_ref, bp1_ref, wp2_ref, bp2_ref, o_ref):
    hb = h_ref[...]
    bb = b_ref[...]
    hg = _gelu(jnp.dot(hb, wg1_ref[...],
                       preferred_element_type=jnp.float32) + bg1_ref[...])
    gate = jnp.dot(hg, wg2_ref[...],
                   preferred_element_type=jnp.float32) + bg2_ref[...]
    mb = bb == lax.broadcasted_iota(jnp.int32, (_N, _G), 1)
    mf = mb.astype(jnp.float32)
    gmax = jnp.max(jnp.where(mb, gate, -1e30), axis=0, keepdims=True)
    gmaxb = lax.dot_general(mf, gmax, (((1,), (1,)), ((), ())),
                            preferred_element_type=jnp.float32)
    att = jnp.exp(gate - gmaxb)
    den = lax.dot_general(mf, att, (((0,), (0,)), ((), ())),
                          preferred_element_type=jnp.float32)
    denb = jnp.dot(mf, den, preferred_element_type=jnp.float32)
    w = att / (denb + 1e-16)
    g = lax.dot_general(mf, w * hb, (((0,), (0,)), ((), ())),
                        preferred_element_type=jnp.float32)
    g = _gelu(jnp.dot(g, wr_ref[...],
                      preferred_element_type=jnp.float32) + br_ref[...])
    g = _gelu(jnp.dot(g, wp1_ref[...],
                      preferred_element_type=jnp.float32) + bp1_ref[...])
    g = jnp.dot(g, wp2_ref[...],
                preferred_element_type=jnp.float32) + bp2_ref[...]
    nrm = jnp.sqrt(jnp.sum(g * g, axis=-1, keepdims=True))
    o_ref[...] = g / (nrm + 1e-12)


def _full(shape):
    return pl.BlockSpec(shape, lambda *_: tuple(0 for _ in shape))


def _sc_body(h_hbm, e_hbm, src_hbm, dst_hbm, out_hbm,
             src_v0, src_v1, dst_v0, dst_v1, hsrc_v0, hsrc_v1, epk0, epk1,
             zv, agg_sh,
             semI0, semI1, semE0, semE1, semG0, semG1, semS0, semS1):
    c = lax.axis_index("c")
    s = lax.axis_index("s")
    wid = s * _NC + c
    zero16 = jnp.zeros((16,), jnp.float32)

    src_v = (src_v0, src_v1)
    dst_v = (dst_v0, dst_v1)
    hsrc_v = (hsrc_v0, hsrc_v1)
    epk = (epk0, epk1)
    semI = (semI0, semI1)
    semE = (semE0, semE1)
    semG = (semG0, semG1)
    semS = (semS0, semS1)

    def zrow(r, carry):
        for j in range(8):
            zv[r, pl.ds(j * 16, 16)] = zero16
        return carry
    lax.fori_loop(0, _ZR, zrow, 0)
    for t in range(_ZPT):
        k = s + t * _NS

        @pl.when(k < _NZCH)
        def _zero_chunk(k=k):
            pltpu.sync_copy(zv, agg_sh.at[pl.ds(k * _ZR, _ZR)])
    plsc.subcore_barrier()

    ebase = wid * _EPW

    _CH2 = _CH // 2

    def _step(k, b, first):
        bn = 1 - b
        if not first:
            pltpu.make_async_copy(hsrc_v[bn], agg_sh.at[dst_v[bn]],
                                  semS[bn]).wait()
        kn = k + 1
        basen = ebase + kn * _CH
        ebasen2 = (wid * _NCHUNK + kn) * _CH2

        @pl.when(kn < _NCHUNK)
        def _prefetch():
            pltpu.async_copy(src_hbm.at[pl.ds(basen, _CH)], src_v[bn],
                             semI[bn])
            pltpu.async_copy(dst_hbm.at[pl.ds(basen, _CH)], dst_v[bn],
                             semI[bn])
            pltpu.async_copy(e_hbm.at[pl.ds(ebasen2, _CH2)], epk[bn],
                             semE[bn])

        pltpu.make_async_copy(h_hbm.at[src_v[b]], hsrc_v[b], semG[b]).wait()
        ebase2 = (wid * _NCHUNK + k) * _CH2
        pltpu.make_async_copy(e_hbm.at[pl.ds(ebase2, _CH2)], epk[b],
                              semE[b]).wait()

        @plsc.parallel_loop(0, _CH2, step=1, unroll=2)
        def _row(rp):
            for half in range(2):
                r = rp + _CH2 * half
                for j in range(4):
                    w = epk[b][rp, pl.ds(_HW * half + j * 16, 16)]
                    elo = lax.bitcast_convert_type(w << 16, jnp.float32)
                    ehi = lax.bitcast_convert_type(w & -65536, jnp.float32)
                    slo = pl.ds(j * 16, 16)
                    shi = pl.ds(_HW + j * 16, 16)
                    hsrc_v[b][r, slo] = jnp.maximum(
                        hsrc_v[b][r, slo] + elo, 0.0)
                    hsrc_v[b][r, shi] = jnp.maximum(
                        hsrc_v[b][r, shi] + ehi, 0.0)
        pltpu.async_copy(hsrc_v[b], agg_sh.at[dst_v[b]], semS[b], add=True)

        @pl.when(kn < _NCHUNK)
        def _next_gather():
            pltpu.make_async_copy(src_hbm.at[pl.ds(basen, _CH)], src_v[bn],
                                  semI[bn]).wait()
            pltpu.make_async_copy(dst_hbm.at[pl.ds(basen, _CH)], dst_v[bn],
                                  semI[bn]).wait()
            pltpu.async_copy(h_hbm.at[src_v[bn]], hsrc_v[bn], semG[bn])

    pltpu.async_copy(src_hbm.at[pl.ds(ebase, _CH)], src_v[0], semI[0])
    pltpu.async_copy(dst_hbm.at[pl.ds(ebase, _CH)], dst_v[0], semI[0])
    pltpu.async_copy(e_hbm.at[pl.ds(wid * _NCHUNK * _CH2, _CH2)], epk[0],
                     semE[0])
    pltpu.make_async_copy(src_hbm.at[pl.ds(ebase, _CH)], src_v[0],
                          semI[0]).wait()
    pltpu.make_async_copy(dst_hbm.at[pl.ds(ebase, _CH)], dst_v[0],
                          semI[0]).wait()
    pltpu.async_copy(h_hbm.at[src_v[0]], hsrc_v[0], semG[0])

    _step(0, 0, True)

    def pair(o, carry):
        _step(1 + 2 * o, 1, False)
        _step(2 + 2 * o, 0, False)
        return carry
    lax.fori_loop(0, (_NCHUNK - 1) // 2, pair, 0)
    pltpu.make_async_copy(hsrc_v[0], agg_sh.at[dst_v[0]], semS[0]).wait()
    plsc.subcore_barrier()

    for t in range(_ZPT):
        k = s + t * _NS

        @pl.when(k < _NZCH)
        def _copy_chunk(k=k):
            pltpu.sync_copy(agg_sh.at[pl.ds(k * _ZR, _ZR)],
                            out_hbm.at[c, pl.ds(k * _ZR, _ZR)])


def _sc_agg(hp, ep, src, dst):
    mesh = plsc.VectorSubcoreMesh(core_axis_name="c", subcore_axis_name="s")
    return pl.kernel(
        _sc_body,
        out_type=jax.ShapeDtypeStruct((_NC, _N, _H), jnp.float32),
        mesh=mesh,
        scratch_types=(
            [pltpu.VMEM((_CH,), jnp.int32)] * 4
            + [pltpu.VMEM((_CH, _H), jnp.float32)] * 2
            + [pltpu.VMEM((_CH // 2, _H), jnp.int32)] * 2
            + [pltpu.VMEM((_ZR, _H), jnp.float32),
               pltpu.VMEM_SHARED((_N, _H), jnp.float32)]
            + [pltpu.SemaphoreType.DMA] * 8
        ),
    )(hp, ep, src, dst)


def kernel(x, edge_index, edge_attr, batch, atom_emb0, atom_emb1, atom_emb2,
           atom_emb3, atom_emb4, atom_emb5, atom_emb6, atom_emb7, atom_emb8,
           edge_emb0, edge_emb1, edge_emb2, W_atom, b_atom, W_edge, b_edge,
           eps, W1, b1, W2, b2, ln_g, ln_b, Wg1, bg1, Wg2, bg2, Wr, br,
           Wp1, bp1, Wp2, bp2):
    atom_embs = [atom_emb0, atom_emb1, atom_emb2, atom_emb3, atom_emb4,
                 atom_emb5, atom_emb6, atom_emb7, atom_emb8]
    emb_a = jnp.concatenate(
        [jnp.pad(t, ((0, p - t.shape[0]), (0, 0)))
         for t, p in zip(atom_embs, _ACP)], axis=0)
    emb_e = jnp.concatenate(
        [jnp.pad(t, ((0, p - t.shape[0]), (0, 0)))
         for t, p in zip([edge_emb0, edge_emb1, edge_emb2], _ECP)], axis=0)

    h = pl.pallas_call(
        _atom_body,
        grid=(5,),
        in_specs=[pl.BlockSpec((_N // 5, 9), lambda i: (i, 0)),
                  _full((_ATOT, 48)), _full((9 * 48, _H)), _full((1, _H))],
        out_specs=pl.BlockSpec((_N // 5, _H), lambda i: (i, 0)),
        out_shape=jax.ShapeDtypeStruct((_N, _H), jnp.float32),
    )(x, emb_a, W_atom, b_atom.reshape(1, -1))

    ep = pl.pallas_call(
        _edge_body,
        grid=(80,),
        in_specs=[pl.BlockSpec((_E // 80, 3), lambda i: (i, 0)),
                  _full((_ETOT, 48)), _full((48, _H)), _full((1, _H))],
        out_specs=pl.BlockSpec((_E // 160, _H), lambda i: (i, 0)),
        out_shape=jax.ShapeDtypeStruct((_E // 2, _H), jnp.int32),
    )(edge_attr, emb_e, W_edge, b_edge.reshape(1, -1))

    src = edge_index[0]
    dst = edge_index[1]

    mlp = pl.pallas_call(
        _mlp_body,
        grid=(5,),
        in_specs=[pl.BlockSpec((_N // 5, _H), lambda i: (i, 0)),
                  pl.BlockSpec((_NC, _N // 5, _H), lambda i: (0, i, 0)),
                  _full((_H, _H)), _full((1, _H)), _full((_H, _H)),
                  _full((1, _H)), _full((1, _H)), _full((1, _H)),
                  _full((1, 1))],
        out_specs=pl.BlockSpec((_N // 5, _H), lambda i: (i, 0)),
        out_shape=jax.ShapeDtypeStruct((_N, _H), jnp.float32),
    )

    for l in range(_L):
        agg = _sc_agg(h, ep, src, dst)
        scale = (1.0 + eps[l]).reshape(1, 1)
        h = mlp(h, agg, W1[l], b1[l].reshape(1, -1), W2[l],
                b2[l].reshape(1, -1), ln_g[l].reshape(1, -1),
                ln_b[l].reshape(1, -1), scale)

    out = pl.pallas_call(
        _pool_body,
        in_specs=[_full((_N, _H)), _full((_N, 1)), _full((_H, _H // 2)),
                  _full((1, _H // 2)), _full((_H // 2, 1)), _full((1, 1)),
                  _full((_H, _H)), _full((1, _H)), _full((_H, _H)),
                  _full((1, _H)), _full((_H, _H)), _full((1, _H))],
        out_specs=_full((_G, _H)),
        out_shape=jax.ShapeDtypeStruct((_G, _H), jnp.float32),
    )(h, batch.reshape(-1, 1), Wg1, bg1.reshape(1, -1), Wg2,
      bg2.reshape(1, -1), Wr, br.reshape(1, -1), Wp1, bp1.reshape(1, -1),
      Wp2, bp2.reshape(1, -1))
    return out


# speedup vs baseline: 1.2473x; 1.1869x over previous
import functools

import jax
import jax.numpy as jnp
from jax import lax
from jax.experimental import pallas as pl
from jax.experimental.pallas import tpu as pltpu
from jax.experimental.pallas import tpu_sc as plsc

_N = 10000
_E = 320000
_H = 128
_L = 4
_G = 400
_HW = _H // 2

_AC = [119, 4, 12, 12, 10, 6, 6, 2, 2]
_ACP = [120, 8, 16, 16, 16, 8, 8, 8, 8]
_AOFF = [0, 120, 128, 144, 160, 176, 184, 192, 200]
_ATOT = 208
_ECP = [8, 8, 8]
_EOFF = [0, 8, 16]
_ETOT = 24

_NC = 2
_NS = 16
_NW = _NC * _NS
_EPW = _E // _NW
_CH = 80
_NCHUNK = _EPW // _CH
_ZR = 40
_NZCH = _N // _ZR
_ZPT = -(-_NZCH // _NS)


def _gelu(v):
    return 0.5 * v * (1.0 + lax.erf(v * 0.7071067811865476))


def _pack_half(v):
    u = lax.bitcast_convert_type(v, jnp.int32)
    rnd = ((u >> 16) & 1) + 32767
    ub = (u + rnd) >> 16
    return (ub[:, :_HW] & 65535) | (ub[:, _HW:] << 16)


def _atom_body(x_ref, emb_ref, w_ref, b_ref, o_ref):
    xb = x_ref[...]
    acc = jnp.zeros((xb.shape[0], _H), jnp.float32) + b_ref[...]
    for i in range(9):
        cp, off = _ACP[i], _AOFF[i]
        ti = jnp.dot(emb_ref[off:off + cp, :], w_ref[48 * i:48 * (i + 1), :],
                     preferred_element_type=jnp.float32)
        oh = (xb[:, i:i + 1] == lax.broadcasted_iota(
            jnp.int32, (xb.shape[0], cp), 1)).astype(jnp.float32)
        acc = acc + jnp.dot(oh, ti, preferred_element_type=jnp.float32)
    o_ref[...] = _gelu(acc)


def _edge_body(a_ref, emb_ref, w_ref, b_ref, o_ref):
    ab = a_ref[...]
    acc = jnp.zeros((ab.shape[0], _H), jnp.float32) + b_ref[...]
    for j in range(3):
        cp, off = _ECP[j], _EOFF[j]
        tj = jnp.dot(emb_ref[off:off + cp, :], w_ref[...],
                     preferred_element_type=jnp.float32)
        oh = (ab[:, j:j + 1] == lax.broadcasted_iota(
            jnp.int32, (ab.shape[0], cp), 1)).astype(jnp.float32)
        acc = acc + jnp.dot(oh, tj, preferred_element_type=jnp.float32)
    p = _pack_half(_gelu(acc))
    p3 = p.reshape(p.shape[0] // _CH, _CH, _HW)
    pp = jnp.concatenate([p3[:, :_CH // 2, :], p3[:, _CH // 2:, :]], axis=2)
    o_ref[...] = pp.reshape(p.shape[0] // 2, _H)


def _mlp_body(h_ref, agg_ref, w1_ref, b1_ref, w2_ref, b2_ref, g_ref, be_ref,
              sc_ref, o_ref):
    hb = h_ref[...]
    outv = hb * sc_ref[...] + agg_ref[0] + agg_ref[1]
    t = jnp.dot(_gelu(jnp.dot(outv, w1_ref[...],
                              preferred_element_type=jnp.float32) + b1_ref[...]),
                w2_ref[...], preferred_element_type=jnp.float32) + b2_ref[...]
    m = jnp.mean(t, axis=-1, keepdims=True)
    var = jnp.mean((t - m) ** 2, axis=-1, keepdims=True)
    tn = (t - m) / jnp.sqrt(var + 1e-5) * g_ref[...] + be_ref[...]
    o_ref[...] = hb + _gelu(tn)


def _pool_body(h_ref, b_ref, wg1_ref, bg1_ref, wg2_ref, bg2_ref, wr_ref,
               br_ref, wp1_ref, bp1_ref, wp2_ref, bp2_ref, o_ref):
    hb = h_ref[...]
    bb = b_ref[...]
    hg = _gelu(jnp.dot(hb, wg1_ref[...],
                       preferred_element_type=jnp.float32) + bg1_ref[...])
    gate = jnp.dot(hg, wg2_ref[...],
                   preferred_element_type=jnp.float32) + bg2_ref[...]
    mb = bb == lax.broadcasted_iota(jnp.int32, (_N, _G), 1)
    mf = mb.astype(jnp.float32)
    gmax = jnp.max(jnp.where(mb, gate, -1e30), axis=0, keepdims=True)
    gmaxb = lax.dot_general(mf, gmax, (((1,), (1,)), ((), ())),
                            preferred_element_type=jnp.float32)
    att = jnp.exp(gate - gmaxb)
    den = lax.dot_general(mf, att, (((0,), (0,)), ((), ())),
                          preferred_element_type=jnp.float32)
    denb = jnp.dot(mf, den, preferred_element_type=jnp.float32)
    w = att / (denb + 1e-16)
    g = lax.dot_general(mf, w * hb, (((0,), (0,)), ((), ())),
                        preferred_element_type=jnp.float32)
    g = _gelu(jnp.dot(g, wr_ref[...],
                      preferred_element_type=jnp.float32) + br_ref[...])
    g = _gelu(jnp.dot(g, wp1_ref[...],
                      preferred_element_type=jnp.float32) + bp1_ref[...])
    g = jnp.dot(g, wp2_ref[...],
                preferred_element_type=jnp.float32) + bp2_ref[...]
    nrm = jnp.sqrt(jnp.sum(g * g, axis=-1, keepdims=True))
    o_ref[...] = g / (nrm + 1e-12)


def _full(shape):
    return pl.BlockSpec(shape, lambda *_: tuple(0 for _ in shape))


def _sc_body(h_hbm, e_hbm, src_hbm, dst_hbm, out_hbm,
             src_v0, src_v1, dst_v0, dst_v1, hsrc_v0, hsrc_v1, epk0, epk1,
             zv, agg_sh,
             semI0, semI1, semJ0, semJ1, semE0, semE1, semG0, semG1,
             semS0, semS1):
    c = lax.axis_index("c")
    s = lax.axis_index("s")
    wid = s * _NC + c
    zero16 = jnp.zeros((16,), jnp.float32)

    src_v = (src_v0, src_v1)
    dst_v = (dst_v0, dst_v1)
    hsrc_v = (hsrc_v0, hsrc_v1)
    epk = (epk0, epk1)
    semI = (semI0, semI1)
    semJ = (semJ0, semJ1)
    semE = (semE0, semE1)
    semG = (semG0, semG1)
    semS = (semS0, semS1)

    def zrow(r, carry):
        for j in range(8):
            zv[r, pl.ds(j * 16, 16)] = zero16
        return carry
    lax.fori_loop(0, _ZR, zrow, 0)
    for t in range(_ZPT):
        k = s + t * _NS

        @pl.when(k < _NZCH)
        def _zero_chunk(k=k):
            pltpu.sync_copy(zv, agg_sh.at[pl.ds(k * _ZR, _ZR)])
    plsc.subcore_barrier()

    ebase = wid * _EPW

    _CH2 = _CH // 2

    def _step(k, b, first):
        bn = 1 - b
        if not first:
            pltpu.make_async_copy(hsrc_v[bn], agg_sh.at[dst_v[bn]],
                                  semS[bn]).wait()
        kn = k + 1
        basen = ebase + kn * _CH
        ebasen2 = (wid * _NCHUNK + kn) * _CH2

        @pl.when(kn < _NCHUNK)
        def _prefetch():
            pltpu.async_copy(dst_hbm.at[pl.ds(basen, _CH)], dst_v[bn],
                             semJ[bn])
            pltpu.async_copy(e_hbm.at[pl.ds(ebasen2, _CH2)], epk[bn],
                             semE[bn])

        pltpu.make_async_copy(h_hbm.at[src_v[b]], hsrc_v[b], semG[b]).wait()
        ebase2 = (wid * _NCHUNK + k) * _CH2
        pltpu.make_async_copy(e_hbm.at[pl.ds(ebase2, _CH2)], epk[b],
                              semE[b]).wait()

        @pl.when(k + 2 < _NCHUNK)
        def _src_prefetch():
            pltpu.async_copy(src_hbm.at[pl.ds(ebase + (k + 2) * _CH, _CH)],
                             src_v[b], semI[b])

        @pl.when(kn < _NCHUNK)
        def _next_gather():
            pltpu.make_async_copy(src_hbm.at[pl.ds(basen, _CH)], src_v[bn],
                                  semI[bn]).wait()
            pltpu.async_copy(h_hbm.at[src_v[bn]], hsrc_v[bn], semG[bn])

        @plsc.parallel_loop(0, _CH2, step=1, unroll=2)
        def _row(rp):
            for half in range(2):
                r = rp + _CH2 * half
                for j in range(4):
                    w = epk[b][rp, pl.ds(_HW * half + j * 16, 16)]
                    elo = lax.bitcast_convert_type(w << 16, jnp.float32)
                    ehi = lax.bitcast_convert_type(w & -65536, jnp.float32)
                    slo = pl.ds(j * 16, 16)
                    shi = pl.ds(_HW + j * 16, 16)
                    hsrc_v[b][r, slo] = jnp.maximum(
                        hsrc_v[b][r, slo] + elo, 0.0)
                    hsrc_v[b][r, shi] = jnp.maximum(
                        hsrc_v[b][r, shi] + ehi, 0.0)
        pltpu.make_async_copy(dst_hbm.at[pl.ds(ebase + k * _CH, _CH)],
                              dst_v[b], semJ[b]).wait()
        pltpu.async_copy(hsrc_v[b], agg_sh.at[dst_v[b]], semS[b], add=True)

    pltpu.async_copy(src_hbm.at[pl.ds(ebase, _CH)], src_v[0], semI[0])
    pltpu.async_copy(src_hbm.at[pl.ds(ebase + _CH, _CH)], src_v[1], semI[1])
    pltpu.async_copy(dst_hbm.at[pl.ds(ebase, _CH)], dst_v[0], semJ[0])
    pltpu.async_copy(e_hbm.at[pl.ds(wid * _NCHUNK * _CH2, _CH2)], epk[0],
                     semE[0])
    pltpu.make_async_copy(src_hbm.at[pl.ds(ebase, _CH)], src_v[0],
                          semI[0]).wait()
    pltpu.async_copy(h_hbm.at[src_v[0]], hsrc_v[0], semG[0])

    _step(0, 0, True)

    def pair(o, carry):
        _step(1 + 2 * o, 1, False)
        _step(2 + 2 * o, 0, False)
        return carry
    lax.fori_loop(0, (_NCHUNK - 1) // 2, pair, 0)
    pltpu.make_async_copy(hsrc_v[0], agg_sh.at[dst_v[0]], semS[0]).wait()
    plsc.subcore_barrier()

    for t in range(_ZPT):
        k = s + t * _NS

        @pl.when(k < _NZCH)
        def _copy_chunk(k=k):
            pltpu.sync_copy(agg_sh.at[pl.ds(k * _ZR, _ZR)],
                            out_hbm.at[c, pl.ds(k * _ZR, _ZR)])


def _sc_agg(hp, ep, src, dst):
    mesh = plsc.VectorSubcoreMesh(core_axis_name="c", subcore_axis_name="s")
    return pl.kernel(
        _sc_body,
        out_type=jax.ShapeDtypeStruct((_NC, _N, _H), jnp.float32),
        mesh=mesh,
        scratch_types=(
            [pltpu.VMEM((_CH,), jnp.int32)] * 4
            + [pltpu.VMEM((_CH, _H), jnp.float32)] * 2
            + [pltpu.VMEM((_CH // 2, _H), jnp.int32)] * 2
            + [pltpu.VMEM((_ZR, _H), jnp.float32),
               pltpu.VMEM_SHARED((_N, _H), jnp.float32)]
            + [pltpu.SemaphoreType.DMA] * 10
        ),
    )(hp, ep, src, dst)


def kernel(x, edge_index, edge_attr, batch, atom_emb0, atom_emb1, atom_emb2,
           atom_emb3, atom_emb4, atom_emb5, atom_emb6, atom_emb7, atom_emb8,
           edge_emb0, edge_emb1, edge_emb2, W_atom, b_atom, W_edge, b_edge,
           eps, W1, b1, W2, b2, ln_g, ln_b, Wg1, bg1, Wg2, bg2, Wr, br,
           Wp1, bp1, Wp2, bp2):
    atom_embs = [atom_emb0, atom_emb1, atom_emb2, atom_emb3, atom_emb4,
                 atom_emb5, atom_emb6, atom_emb7, atom_emb8]
    emb_a = jnp.concatenate(
        [jnp.pad(t, ((0, p - t.shape[0]), (0, 0)))
         for t, p in zip(atom_embs, _ACP)], axis=0)
    emb_e = jnp.concatenate(
        [jnp.pad(t, ((0, p - t.shape[0]), (0, 0)))
         for t, p in zip([edge_emb0, edge_emb1, edge_emb2], _ECP)], axis=0)

    h = pl.pallas_call(
        _atom_body,
        grid=(5,),
        in_specs=[pl.BlockSpec((_N // 5, 9), lambda i: (i, 0)),
                  _full((_ATOT, 48)), _full((9 * 48, _H)), _full((1, _H))],
        out_specs=pl.BlockSpec((_N // 5, _H), lambda i: (i, 0)),
        out_shape=jax.ShapeDtypeStruct((_N, _H), jnp.float32),
    )(x, emb_a, W_atom, b_atom.reshape(1, -1))

    ep = pl.pallas_call(
        _edge_body,
        grid=(80,),
        in_specs=[pl.BlockSpec((_E // 80, 3), lambda i: (i, 0)),
                  _full((_ETOT, 48)), _full((48, _H)), _full((1, _H))],
        out_specs=pl.BlockSpec((_E // 160, _H), lambda i: (i, 0)),
        out_shape=jax.ShapeDtypeStruct((_E // 2, _H), jnp.int32),
    )(edge_attr, emb_e, W_edge, b_edge.reshape(1, -1))

    src = edge_index[0]
    dst = edge_index[1]

    mlp = pl.pallas_call(
        _mlp_body,
        grid=(5,),
        in_specs=[pl.BlockSpec((_N // 5, _H), lambda i: (i, 0)),
                  pl.BlockSpec((_NC, _N // 5, _H), lambda i: (0, i, 0)),
                  _full((_H, _H)), _full((1, _H)), _full((_H, _H)),
                  _full((1, _H)), _full((1, _H)), _full((1, _H)),
                  _full((1, 1))],
        out_specs=pl.BlockSpec((_N // 5, _H), lambda i: (i, 0)),
        out_shape=jax.ShapeDtypeStruct((_N, _H), jnp.float32),
    )

    for l in range(_L):
        agg = _sc_agg(h, ep, src, dst)
        scale = (1.0 + eps[l]).reshape(1, 1)
        h = mlp(h, agg, W1[l], b1[l].reshape(1, -1), W2[l],
                b2[l].reshape(1, -1), ln_g[l].reshape(1, -1),
                ln_b[l].reshape(1, -1), scale)

    out = pl.pallas_call(
        _pool_body,
        in_specs=[_full((_N, _H)), _full((_N, 1)), _full((_H, _H // 2)),
                  _full((1, _H // 2)), _full((_H // 2, 1)), _full((1, 1)),
                  _full((_H, _H)), _full((1, _H)), _full((_H, _H)),
                  _full((1, _H)), _full((_H, _H)), _full((1, _H))],
        out_specs=_full((_G, _H)),
        out_shape=jax.ShapeDtypeStruct((_G, _H), jnp.float32),
    )(h, batch.reshape(-1, 1), Wg1, bg1.reshape(1, -1), Wg2,
      bg2.reshape(1, -1), Wr, br.reshape(1, -1), Wp1, bp1.reshape(1, -1),
      Wp2, bp2.reshape(1, -1))
    return out
